# baseline (speedup 1.0000x reference)
import functools

import jax
import jax.numpy as jnp
from jax import lax
from jax.experimental import pallas as pl
from jax.experimental.pallas import tpu as pltpu
from jax.experimental.pallas import tpu_sc as plsc

PRE_SEQ_LEN = 128
OUT_DIM = 12 * 2 * 768
BATCH = 32

NC = 2
NS = 16

SP_TOTAL = 14336
SP_COLS = SP_TOTAL // NC
TL_GROUPS = 8
TL_COLS = (OUT_DIM - SP_TOTAL) // TL_GROUPS
TL_WAYS = NC * NS // TL_GROUPS
TL_BATCHES = BATCH // TL_WAYS
BATCHES_PER_TILE = BATCH // NS
TAB_ROWS_PER_TILE = PRE_SEQ_LEN // NS


def _body(prefix_hbm, table_hbm, out_hbm, spmem, tab_v, idx_v, sem_sp, sem_tl):
    c = lax.axis_index("c")
    s = lax.axis_index("s")
    sp0 = c * SP_COLS
    g = s % TL_GROUPS
    q = c * 2 + s // TL_GROUPS
    tl0 = SP_TOTAL + g * TL_COLS
    b0 = q * TL_BATCHES

    pltpu.sync_copy(prefix_hbm, idx_v)
    pltpu.sync_copy(table_hbm.at[:, pl.ds(tl0, TL_COLS)], tab_v)
    tr0 = s * TAB_ROWS_PER_TILE
    pltpu.sync_copy(
        table_hbm.at[pl.ds(tr0, TAB_ROWS_PER_TILE), pl.ds(sp0, SP_COLS)],
        spmem.at[pl.ds(tr0, TAB_ROWS_PER_TILE), :],
    )
    plsc.subcore_barrier()

    def issue_sp(k, carry):
        r0 = k * 16
        batch = s * BATCHES_PER_TILE + r0 // PRE_SEQ_LEN
        row0 = r0 % PRE_SEQ_LEN
        v = idx_v[batch, pl.ds(row0, 16)]
        for j in range(16):
            pltpu.async_copy(
                spmem.at[v[j]],
                out_hbm.at[batch, row0 + j, pl.ds(sp0, SP_COLS)],
                sem_sp,
            )
        return carry

    n_sp = BATCHES_PER_TILE * PRE_SEQ_LEN // 16
    lax.fori_loop(0, n_sp, issue_sp, 0)

    def issue_tl(k, carry):
        batch = b0 + k // (PRE_SEQ_LEN // 16)
        row0 = (k % (PRE_SEQ_LEN // 16)) * 16
        v = idx_v[batch, pl.ds(row0, 16)]
        for j in range(16):
            pltpu.async_copy(
                tab_v.at[v[j]],
                out_hbm.at[batch, row0 + j, pl.ds(tl0, TL_COLS)],
                sem_tl,
            )
        return carry

    n_tl = TL_BATCHES * PRE_SEQ_LEN // 16
    lax.fori_loop(0, n_tl, issue_tl, 0)

    def drain_sp(k, carry):
        pltpu.make_async_copy(
            spmem.at[0],
            out_hbm.at[s * BATCHES_PER_TILE, 0, pl.ds(sp0, SP_COLS)],
            sem_sp,
        ).wait()
        return carry

    lax.fori_loop(0, n_sp * 16, drain_sp, 0)

    def drain_tl(k, carry):
        pltpu.make_async_copy(
            tab_v.at[0],
            out_hbm.at[b0, 0, pl.ds(tl0, TL_COLS)],
            sem_tl,
        ).wait()
        return carry

    lax.fori_loop(0, n_tl * 16, drain_tl, 0)


_gather = functools.partial(
    pl.kernel,
    out_type=jax.ShapeDtypeStruct((BATCH, PRE_SEQ_LEN, OUT_DIM), jnp.float32),
    mesh=plsc.VectorSubcoreMesh(core_axis_name="c", subcore_axis_name="s"),
    scratch_types=[
        pltpu.VMEM_SHARED((PRE_SEQ_LEN, SP_COLS), jnp.float32),
        pltpu.VMEM((PRE_SEQ_LEN, TL_COLS), jnp.float32),
        pltpu.VMEM((BATCH, PRE_SEQ_LEN), jnp.int32),
        pltpu.SemaphoreType.DMA,
        pltpu.SemaphoreType.DMA,
    ],
    compiler_params=pltpu.CompilerParams(use_tc_tiling_on_sc=True),
)(_body)


@jax.jit
def kernel(prefix, emb_table):
    return _gather(prefix.astype(jnp.int32), emb_table)


# speedup vs baseline: 1.0605x; 1.0605x over previous
import functools

import jax
import jax.numpy as jnp
from jax import lax
from jax.experimental import pallas as pl
from jax.experimental.pallas import tpu as pltpu
from jax.experimental.pallas import tpu_sc as plsc

PRE_SEQ_LEN = 128
OUT_DIM = 12 * 2 * 768
BATCH = 32

NC = 2
NS = 16

SP_TOTAL = 12288
SP_COLS = SP_TOTAL // NC
TL_COLS = (OUT_DIM - SP_TOTAL) // NS
TL_BATCHES = BATCH // NC
BATCHES_PER_TILE = BATCH // NS
TAB_ROWS_PER_TILE = PRE_SEQ_LEN // NS


def _body(prefix_hbm, table_hbm, out_hbm, spmem, tab_v, idx_v, sem_sp, sem_tl):
    c = lax.axis_index("c")
    s = lax.axis_index("s")
    sp0 = c * SP_COLS
    tl0 = SP_TOTAL + s * TL_COLS
    b0 = c * TL_BATCHES

    pltpu.sync_copy(prefix_hbm, idx_v)
    pltpu.sync_copy(table_hbm.at[:, pl.ds(tl0, TL_COLS)], tab_v)

    def issue_tl(k, carry):
        batch = b0 + k // (PRE_SEQ_LEN // 16)
        row0 = (k % (PRE_SEQ_LEN // 16)) * 16
        v = idx_v[batch, pl.ds(row0, 16)]
        for j in range(16):
            pltpu.async_copy(
                tab_v.at[v[j]],
                out_hbm.at[batch, row0 + j, pl.ds(tl0, TL_COLS)],
                sem_tl,
            )
        return carry

    n_tl = TL_BATCHES * PRE_SEQ_LEN // 16
    lax.fori_loop(0, n_tl, issue_tl, 0)

    tr0 = s * TAB_ROWS_PER_TILE
    pltpu.sync_copy(
        table_hbm.at[pl.ds(tr0, TAB_ROWS_PER_TILE), pl.ds(sp0, SP_COLS)],
        spmem.at[pl.ds(tr0, TAB_ROWS_PER_TILE), :],
    )
    plsc.subcore_barrier()

    def issue_sp(k, carry):
        r0 = k * 16
        batch = s * BATCHES_PER_TILE + r0 // PRE_SEQ_LEN
        row0 = r0 % PRE_SEQ_LEN
        v = idx_v[batch, pl.ds(row0, 16)]
        for j in range(16):
            pltpu.async_copy(
                spmem.at[v[j]],
                out_hbm.at[batch, row0 + j, pl.ds(sp0, SP_COLS)],
                sem_sp,
            )
        return carry

    n_sp = BATCHES_PER_TILE * PRE_SEQ_LEN // 16
    lax.fori_loop(0, n_sp, issue_sp, 0)

    def drain_tl(k, carry):
        pltpu.make_async_copy(
            tab_v.at[0],
            out_hbm.at[b0, 0, pl.ds(tl0, TL_COLS)],
            sem_tl,
        ).wait()
        return carry

    lax.fori_loop(0, n_tl * 16, drain_tl, 0)

    def drain_sp(k, carry):
        pltpu.make_async_copy(
            spmem.at[0],
            out_hbm.at[s * BATCHES_PER_TILE, 0, pl.ds(sp0, SP_COLS)],
            sem_sp,
        ).wait()
        return carry

    lax.fori_loop(0, n_sp * 16, drain_sp, 0)


_gather = functools.partial(
    pl.kernel,
    out_type=jax.ShapeDtypeStruct((BATCH, PRE_SEQ_LEN, OUT_DIM), jnp.float32),
    mesh=plsc.VectorSubcoreMesh(core_axis_name="c", subcore_axis_name="s"),
    scratch_types=[
        pltpu.VMEM_SHARED((PRE_SEQ_LEN, SP_COLS), jnp.float32),
        pltpu.VMEM((PRE_SEQ_LEN, TL_COLS), jnp.float32),
        pltpu.VMEM((BATCH, PRE_SEQ_LEN), jnp.int32),
        pltpu.SemaphoreType.DMA,
        pltpu.SemaphoreType.DMA,
    ],
    compiler_params=pltpu.CompilerParams(use_tc_tiling_on_sc=True),
)(_body)


@jax.jit
def kernel(prefix, emb_table):
    return _gather(prefix.astype(jnp.int32), emb_table)
